# baseline (speedup 1.0000x reference)
import functools

import jax
import jax.numpy as jnp
from jax import lax
from jax.experimental import pallas as pl
from jax.experimental.pallas import tpu as pltpu
from jax.experimental.pallas import tpu_sc as plsc

N_NODES = 10000
N_PAD = 10112
N_EDGES = 160000
E_PAD = 160256
D_HID = 16
NS = 16
E_PER_TILE = E_PAD // NS
E_CHUNK = E_PER_TILE // 2
ROWS_PER_TILE = N_PAD // NS


def _splat_lane(vec, j):
    return lax.gather(
        vec, jnp.full((D_HID, 1), j, jnp.int32),
        dimension_numbers=lax.GatherDimensionNumbers(
            offset_dims=(), collapsed_slice_dims=(0,), start_index_map=(0,)),
        slice_sizes=(1,),
        mode=lax.GatherScatterMode.PROMISE_IN_BOUNDS)


def _make_sc_gcn():
    mesh = plsc.VectorSubcoreMesh(core_axis_name="c", subcore_axis_name="s",
                                  num_cores=1)

    @functools.partial(
        pl.kernel,
        mesh=mesh,
        compiler_params=pltpu.CompilerParams(use_tc_tiling_on_sc=False),
        out_type=jax.ShapeDtypeStruct((N_PAD, D_HID), jnp.float32),
        scratch_types=[
            pltpu.VMEM((E_CHUNK,), jnp.int32),
            pltpu.VMEM((E_CHUNK,), jnp.int32),
            pltpu.VMEM((E_CHUNK,), jnp.float32),
            pltpu.VMEM((E_CHUNK, D_HID), jnp.float32),
            pltpu.VMEM((ROWS_PER_TILE, D_HID), jnp.float32),
            pltpu.VMEM((D_HID,), jnp.float32),
            pltpu.VMEM_SHARED((N_PAD, D_HID), jnp.float32),
            pltpu.VMEM_SHARED((N_PAD, D_HID), jnp.float32),
            pltpu.SemaphoreType.DMA,
        ],
    )
    def gcn(rows_hbm, cols_hbm, vals_hbm, h1_hbm, b_hbm, out_hbm,
            rows_v, cols_v, vals_v, gath_v, zbuf_v, b_v, acc1_sh, acc2_sh,
            sem):
        s = lax.axis_index("s")
        my_rows = pl.ds(s * ROWS_PER_TILE, ROWS_PER_TILE)

        def zero_body(j, _):
            zbuf_v[j, :] = jnp.zeros((D_HID,), jnp.float32)
            return 0
        lax.fori_loop(0, ROWS_PER_TILE, zero_body, 0)
        pltpu.sync_copy(zbuf_v, acc1_sh.at[my_rows])
        pltpu.sync_copy(zbuf_v, acc2_sh.at[my_rows])
        pltpu.sync_copy(b_hbm, b_v)
        bias = b_v[:]
        plsc.subcore_barrier()

        def spmm_chunk(k, src_hbm_or_sh, acc_sh):
            base = s * E_PER_TILE + k * E_CHUNK
            cp_r = pltpu.async_copy(
                rows_hbm.at[pl.ds(base, E_CHUNK)], rows_v, sem)
            cp_c = pltpu.async_copy(
                cols_hbm.at[pl.ds(base, E_CHUNK)], cols_v, sem)
            cp_v = pltpu.async_copy(
                vals_hbm.at[pl.ds(base, E_CHUNK)], vals_v, sem)
            cp_r.wait()
            cp_c.wait()
            cp_v.wait()
            pltpu.async_copy(src_hbm_or_sh.at[cols_v], gath_v, sem).wait()

            def scale_body(g, _):
                vals16 = vals_v[pl.ds(g * 16, 16)]
                for j in range(16):
                    e = g * 16 + j
                    gath_v[e, :] = gath_v[e, :] * _splat_lane(vals16, j)
                return 0
            lax.fori_loop(0, E_CHUNK // 16, scale_body, 0)
            pltpu.sync_copy(gath_v, acc_sh.at[rows_v], add=True)

        for k in range(2):
            spmm_chunk(k, h1_hbm, acc1_sh)
        plsc.subcore_barrier()

        pltpu.sync_copy(acc1_sh.at[my_rows], zbuf_v)

        def relu_body(j, _):
            zbuf_v[j, :] = jnp.maximum(zbuf_v[j, :] + bias, 0.0)
            return 0
        lax.fori_loop(0, ROWS_PER_TILE, relu_body, 0)
        pltpu.sync_copy(zbuf_v, acc1_sh.at[my_rows])
        plsc.subcore_barrier()

        for k in range(2):
            spmm_chunk(k, acc1_sh, acc2_sh)
        plsc.subcore_barrier()

        pltpu.sync_copy(acc2_sh.at[my_rows], out_hbm.at[my_rows])

    return gcn


_sc_gcn = _make_sc_gcn()


def _tc_matmul1(feature, w1):
    m, k = feature.shape
    n = w1.shape[1]
    bm = 632

    def body(x_ref, w_ref, o_ref):
        o_ref[:] = jnp.dot(x_ref[:], w_ref[:],
                           preferred_element_type=jnp.float32)

    return pl.pallas_call(
        body,
        grid=(m // bm,),
        in_specs=[pl.BlockSpec((bm, k), lambda i: (i, 0)),
                  pl.BlockSpec((k, n), lambda i: (0, 0))],
        out_specs=pl.BlockSpec((bm, n), lambda i: (i, 0)),
        out_shape=jax.ShapeDtypeStruct((m, n), jnp.float32),
    )(feature, w1)


def _tc_final(q, w2, b2):
    m, k = q.shape
    n = w2.shape[1]
    bm = 632

    def body(q_ref, w_ref, b_ref, o_ref):
        o_ref[:] = jnp.dot(q_ref[:], w_ref[:],
                           preferred_element_type=jnp.float32) + b_ref[:]

    return pl.pallas_call(
        body,
        grid=(m // bm,),
        in_specs=[pl.BlockSpec((bm, k), lambda i: (i, 0)),
                  pl.BlockSpec((k, n), lambda i: (0, 0)),
                  pl.BlockSpec((1, n), lambda i: (0, 0))],
        out_specs=pl.BlockSpec((bm, n), lambda i: (i, 0)),
        out_shape=jax.ShapeDtypeStruct((m, n), jnp.float32),
    )(q, w2, b2)


def kernel(adjacency_edge_index, adjacency_values, feature, W1, b1, W2, b2):
    epad = E_PAD - N_EDGES
    rows = jnp.pad(adjacency_edge_index[0], (0, epad))
    cols = jnp.pad(adjacency_edge_index[1], (0, epad))
    vals = jnp.pad(adjacency_values, (0, epad))
    feature_p = jnp.pad(feature, ((0, N_PAD - N_NODES), (0, 0)))

    h1 = _tc_matmul1(feature_p, W1)
    q = _sc_gcn(rows, cols, vals, h1, b1)
    return _tc_final(q, W2, b2.reshape(1, -1))[:N_NODES]
